# baseline (speedup 1.0000x reference)
import functools

import jax
import jax.numpy as jnp
from jax import lax
from jax.experimental import pallas as pl
from jax.experimental.pallas import tpu as pltpu
from jax.experimental.pallas import tpu_sc as plsc

F = 128
B = 16384
NW = 32
BPW = B // NW
CH = 32
NCH = BPW // CH
NBUF = 4
L = 16
RG = 4


def _body(user_hbm, rsid_hbm, ut_hbm, rt_hbm, out_hbm,
          u_idx, r_idx, u_rows, r_rows, out_v,
          sem_i, sem_u0, sem_u1, sem_r0, sem_r1):
    wid = lax.axis_index("s") * 2 + lax.axis_index("c")
    base = wid * BPW

    isems = {0: (sem_u0, sem_r0), 1: (sem_u1, sem_r1)}
    early, rest = [], []
    for c in range(NCH):
        su, sr = isems.get(c, (sem_i, sem_i))
        lst = early if c < 2 else rest
        lst.append(pltpu.async_copy(
            user_hbm.at[pl.ds(base + c * CH, CH)], u_idx.at[c], su))
        lst.append(pltpu.async_copy(
            rsid_hbm.at[pl.ds(base + c * CH, CH)], r_idx.at[c], sr))

    def start(c, par):
        su = sem_u0 if par == 0 else sem_u1
        sr = sem_r0 if par == 0 else sem_r1
        buf = c % NBUF
        cu = pltpu.async_copy(ut_hbm.at[u_idx.at[c]], u_rows.at[buf], su)
        cr = pltpu.async_copy(rt_hbm.at[r_idx.at[c]], r_rows.at[buf], sr)
        return cu, cr

    lanes = lax.iota(jnp.int32, L)
    perms = {k: lanes ^ k for k in (8, 4, 2, 1)}
    dnums = lax.GatherDimensionNumbers(
        offset_dims=(), collapsed_slice_dims=(0,), start_index_map=(0,))

    def perm(x, k):
        return lax.gather(
            x, perms[k][:, None], dimension_numbers=dnums, slice_sizes=(1,),
            mode=lax.GatherScatterMode.PROMISE_IN_BOUNDS)

    def lane_sum(x):
        for k in (8, 4, 2, 1):
            x = x + perm(x, k)
        return x

    def row_acc(buf, row):
        prods = [u_rows[buf, row, pl.ds(fs * L, L)]
                 * r_rows[buf, row, pl.ds(fs * L, L)]
                 for fs in range(F // L)]
        while len(prods) > 1:
            prods = [a + b for a, b in zip(prods[::2], prods[1::2])]
        return prods[0]

    def run_chunk(c):
        buf = c % NBUF
        ngrp = CH // RG

        def group_body(j, carry):
            accs, outv = carry
            lb = ((j - 1) % (L // RG)) * RG
            for i in range(RG):
                s = lane_sum(accs[i])
                outv = jnp.where(lanes == lb + i, s, outv)

            row0 = jnp.minimum(j, ngrp - 1) * RG
            accs = tuple(row_acc(buf, row0 + i) for i in range(RG))

            off = ((jnp.maximum(j, 1) - 1) // (L // RG)) * L
            out_v[pl.ds(c * CH + off, L)] = outv
            return accs, outv

        zero = jnp.zeros((L,), jnp.float32)
        lax.fori_loop(0, ngrp + 1, group_body, ((zero,) * RG, zero))

    def wait_chunk(c, par):
        su = sem_u0 if par == 0 else sem_u1
        sr = sem_r0 if par == 0 else sem_r1
        buf = c % NBUF
        pltpu.make_async_copy(ut_hbm.at[u_idx.at[c]], u_rows.at[buf],
                              su).wait()
        pltpu.make_async_copy(rt_hbm.at[r_idx.at[c]], r_rows.at[buf],
                              sr).wait()

    early[0].wait()
    early[1].wait()
    start(0, 0)
    early[2].wait()
    early[3].wait()
    start(1, 1)
    for h in rest:
        h.wait()

    def pair_body(p, _):
        c0 = p * 2
        wait_chunk(c0, 0)

        @pl.when(c0 + 2 < NCH)
        def _prefetch_even():
            start(c0 + 2, 0)
        run_chunk(c0)

        wait_chunk(c0 + 1, 1)

        @pl.when(c0 + 3 < NCH)
        def _prefetch_odd():
            start(c0 + 3, 1)
        run_chunk(c0 + 1)
        return 0

    lax.fori_loop(0, NCH // 2, pair_body, 0)

    pltpu.sync_copy(out_v, out_hbm.at[pl.ds(base, BPW)])


@jax.jit
def _run(user, rsid, user_table, rsid_table):
    mesh = plsc.VectorSubcoreMesh(core_axis_name="c", subcore_axis_name="s")
    k = functools.partial(
        pl.kernel,
        out_type=jax.ShapeDtypeStruct((B,), jnp.float32),
        mesh=mesh,
        scratch_types=[
            pltpu.VMEM((NCH, CH), jnp.int32),
            pltpu.VMEM((NCH, CH), jnp.int32),
            pltpu.VMEM((NBUF, CH, F), jnp.float32),
            pltpu.VMEM((NBUF, CH, F), jnp.float32),
            pltpu.VMEM((BPW,), jnp.float32),
            pltpu.SemaphoreType.DMA,
            pltpu.SemaphoreType.DMA,
            pltpu.SemaphoreType.DMA,
            pltpu.SemaphoreType.DMA,
            pltpu.SemaphoreType.DMA,
        ],
    )(_body)
    return k(user, rsid, user_table, rsid_table)


def kernel(user, rsid, user_table, rsid_table):
    return _run(user, rsid, user_table, rsid_table)


# speedup vs baseline: 1.0573x; 1.0573x over previous
import functools

import jax
import jax.numpy as jnp
from jax import lax
from jax.experimental import pallas as pl
from jax.experimental.pallas import tpu as pltpu
from jax.experimental.pallas import tpu_sc as plsc

F = 128
B = 16384
NW = 32
BPW = B // NW
CH = 64
NCH = BPW // CH
NBUF = 4
L = 16
RG = 4


def _body(user_hbm, rsid_hbm, ut_hbm, rt_hbm, out_hbm,
          u_idx, r_idx, u_rows, r_rows, out_v,
          sem_i, sem_u0, sem_u1, sem_r0, sem_r1):
    wid = lax.axis_index("s") * 2 + lax.axis_index("c")
    base = wid * BPW

    isems = {0: (sem_u0, sem_r0), 1: (sem_u1, sem_r1)}
    early, rest = [], []
    for c in range(NCH):
        su, sr = isems.get(c, (sem_i, sem_i))
        lst = early if c < 2 else rest
        lst.append(pltpu.async_copy(
            user_hbm.at[pl.ds(base + c * CH, CH)], u_idx.at[c], su))
        lst.append(pltpu.async_copy(
            rsid_hbm.at[pl.ds(base + c * CH, CH)], r_idx.at[c], sr))

    def start(c, par):
        su = sem_u0 if par == 0 else sem_u1
        sr = sem_r0 if par == 0 else sem_r1
        buf = c % NBUF
        cu = pltpu.async_copy(ut_hbm.at[u_idx.at[c]], u_rows.at[buf], su)
        cr = pltpu.async_copy(rt_hbm.at[r_idx.at[c]], r_rows.at[buf], sr)
        return cu, cr

    lanes = lax.iota(jnp.int32, L)
    perms = {k: lanes ^ k for k in (8, 4, 2, 1)}
    dnums = lax.GatherDimensionNumbers(
        offset_dims=(), collapsed_slice_dims=(0,), start_index_map=(0,))

    def perm(x, k):
        return lax.gather(
            x, perms[k][:, None], dimension_numbers=dnums, slice_sizes=(1,),
            mode=lax.GatherScatterMode.PROMISE_IN_BOUNDS)

    def lane_sum(x):
        for k in (8, 4, 2, 1):
            x = x + perm(x, k)
        return x

    def row_acc(buf, row):
        prods = [u_rows[buf, row, pl.ds(fs * L, L)]
                 * r_rows[buf, row, pl.ds(fs * L, L)]
                 for fs in range(F // L)]
        while len(prods) > 1:
            prods = [a + b for a, b in zip(prods[::2], prods[1::2])]
        return prods[0]

    def run_chunk(c):
        buf = c % NBUF
        ngrp = CH // RG

        def group_body(j, carry):
            accs, outv = carry
            lb = ((j - 1) % (L // RG)) * RG
            for i in range(RG):
                s = lane_sum(accs[i])
                outv = jnp.where(lanes == lb + i, s, outv)

            row0 = jnp.minimum(j, ngrp - 1) * RG
            accs = tuple(row_acc(buf, row0 + i) for i in range(RG))

            off = ((jnp.maximum(j, 1) - 1) // (L // RG)) * L
            out_v[pl.ds(c * CH + off, L)] = outv
            return accs, outv

        zero = jnp.zeros((L,), jnp.float32)
        lax.fori_loop(0, ngrp + 1, group_body, ((zero,) * RG, zero))

    def wait_chunk(c, par):
        su = sem_u0 if par == 0 else sem_u1
        sr = sem_r0 if par == 0 else sem_r1
        buf = c % NBUF
        pltpu.make_async_copy(ut_hbm.at[u_idx.at[c]], u_rows.at[buf],
                              su).wait()
        pltpu.make_async_copy(rt_hbm.at[r_idx.at[c]], r_rows.at[buf],
                              sr).wait()

    early[0].wait()
    early[1].wait()
    start(0, 0)
    early[2].wait()
    early[3].wait()
    start(1, 1)
    for h in rest:
        h.wait()

    def pair_body(p, _):
        c0 = p * 2
        wait_chunk(c0, 0)

        @pl.when(c0 + 2 < NCH)
        def _prefetch_even():
            start(c0 + 2, 0)
        run_chunk(c0)

        wait_chunk(c0 + 1, 1)

        @pl.when(c0 + 3 < NCH)
        def _prefetch_odd():
            start(c0 + 3, 1)
        run_chunk(c0 + 1)
        return 0

    lax.fori_loop(0, NCH // 2, pair_body, 0)

    pltpu.sync_copy(out_v, out_hbm.at[pl.ds(base, BPW)])


@jax.jit
def _run(user, rsid, user_table, rsid_table):
    mesh = plsc.VectorSubcoreMesh(core_axis_name="c", subcore_axis_name="s")
    k = functools.partial(
        pl.kernel,
        out_type=jax.ShapeDtypeStruct((B,), jnp.float32),
        mesh=mesh,
        scratch_types=[
            pltpu.VMEM((NCH, CH), jnp.int32),
            pltpu.VMEM((NCH, CH), jnp.int32),
            pltpu.VMEM((NBUF, CH, F), jnp.float32),
            pltpu.VMEM((NBUF, CH, F), jnp.float32),
            pltpu.VMEM((BPW,), jnp.float32),
            pltpu.SemaphoreType.DMA,
            pltpu.SemaphoreType.DMA,
            pltpu.SemaphoreType.DMA,
            pltpu.SemaphoreType.DMA,
            pltpu.SemaphoreType.DMA,
        ],
    )(_body)
    return k(user, rsid, user_table, rsid_table)


def kernel(user, rsid, user_table, rsid_table):
    return _run(user, rsid, user_table, rsid_table)
